# baseline (speedup 1.0000x reference)
import functools

import jax
import jax.numpy as jnp
from jax import lax
from jax.experimental import pallas as pl
from jax.experimental.pallas import tpu as pltpu
from jax.experimental.pallas import tpu_sc as plsc

_MARGIN = 1.0
_L = 16

_DNUMS = lax.GatherDimensionNumbers(
    offset_dims=(), collapsed_slice_dims=(0,), start_index_map=(0,))


def _triplet_kernel(T, B, D, NW, TPW):
    mesh = plsc.VectorSubcoreMesh(core_axis_name="c", subcore_axis_name="s")

    @functools.partial(
        pl.kernel,
        mesh=mesh,
        out_type=jax.ShapeDtypeStruct((NW, _L), jnp.float32),
        compiler_params=pltpu.CompilerParams(use_tc_tiling_on_sc=False),
        scratch_types=[
            pltpu.VMEM((TPW,), jnp.int32),
            pltpu.VMEM((TPW,), jnp.int32),
            pltpu.VMEM((TPW,), jnp.int32),
            pltpu.VMEM((TPW, D), jnp.float32),
            pltpu.VMEM((TPW, D), jnp.float32),
            pltpu.VMEM((TPW, D), jnp.float32),
            pltpu.VMEM((_L,), jnp.float32),
            pltpu.SemaphoreType.DMA,
        ],
    )
    def k(emb_hbm, ia_hbm, ip_hbm, in_hbm, out_hbm, ia_v, ip_v, in_v,
          a_v, p_v, n_v, out_v, sem):
        wid = lax.axis_index("s") * 2 + lax.axis_index("c")
        base = wid * TPW

        pltpu.sync_copy(ia_hbm.at[pl.ds(base, TPW)], ia_v)
        pltpu.sync_copy(ip_hbm.at[pl.ds(base, TPW)], ip_v)
        pltpu.sync_copy(in_hbm.at[pl.ds(base, TPW)], in_v)

        copies = [
            pltpu.make_async_copy(emb_hbm.at[ia_v], a_v, sem),
            pltpu.make_async_copy(emb_hbm.at[ip_v], p_v, sem),
            pltpu.make_async_copy(emb_hbm.at[in_v], n_v, sem),
        ]
        for cpy in copies:
            cpy.start()
        for cpy in copies:
            cpy.wait()

        lanes = lax.iota(jnp.int32, _L)
        perm_idx = {sh: (lanes ^ sh)[:, None] for sh in (1, 2, 4, 8)}

        def permute(v, sh):
            return lax.gather(v, perm_idx[sh], _DNUMS, (1,),
                              mode=lax.GatherScatterMode.PROMISE_IN_BOUNDS)

        def group(g, acc):
            t0 = g * _L
            vts = []
            for j in range(_L):
                t = t0 + j
                lane = None
                for c in range(D // _L):
                    sl = pl.ds(c * _L, _L)
                    a = a_v[t, sl]
                    p = p_v[t, sl]
                    n = n_v[t, sl]
                    pn = p - n
                    q = (p + n) - a - a
                    lane = pn * q if lane is None else lane + pn * q
                vts.append(lane)
            sh = 1
            while len(vts) > 1:
                nxt = []
                for i in range(0, len(vts), 2):
                    x, y = vts[i], vts[i + 1]
                    r = jnp.where((lanes & sh) == 0,
                                  x + permute(x, sh), y + permute(y, sh))
                    nxt.append(r)
                vts = nxt
                sh *= 2
            return acc + jnp.maximum(vts[0] + _MARGIN, 0.0)

        acc = lax.fori_loop(0, TPW // _L, group, jnp.zeros((_L,), jnp.float32))

        for sh in (8, 4, 2, 1):
            acc = acc + permute(acc, sh)
        out_v[...] = acc
        pltpu.sync_copy(out_v, out_hbm.at[wid])

    return k


def kernel(embeddings, target, triplets):
    del target
    T = triplets.shape[0]
    B, D = embeddings.shape
    NW = 32
    TPW = T // NW
    ia = triplets[:, 0]
    ip = triplets[:, 1]
    inn = triplets[:, 2]
    partials = _triplet_kernel(T, B, D, NW, TPW)(embeddings, ia, ip, inn)
    return (jnp.sum(partials[:, 0]) / T, T)


# speedup vs baseline: 1.3246x; 1.3246x over previous
import functools

import jax
import jax.numpy as jnp
from jax import lax
from jax.experimental import pallas as pl
from jax.experimental.pallas import tpu as pltpu
from jax.experimental.pallas import tpu_sc as plsc

_MARGIN = 1.0
_L = 16

_DNUMS = lax.GatherDimensionNumbers(
    offset_dims=(), collapsed_slice_dims=(0,), start_index_map=(0,))


def _triplet_kernel(T, B, D, NW, TPW):
    mesh = plsc.VectorSubcoreMesh(core_axis_name="c", subcore_axis_name="s")

    @functools.partial(
        pl.kernel,
        mesh=mesh,
        out_type=jax.ShapeDtypeStruct((NW, _L), jnp.float32),
        compiler_params=pltpu.CompilerParams(use_tc_tiling_on_sc=False),
        scratch_types=[
            pltpu.VMEM((TPW,), jnp.int32),
            pltpu.VMEM((TPW,), jnp.int32),
            pltpu.VMEM((TPW,), jnp.int32),
            pltpu.VMEM((TPW, D), jnp.float32),
            pltpu.VMEM((TPW, D), jnp.float32),
            pltpu.VMEM((TPW, D), jnp.float32),
            pltpu.VMEM((_L,), jnp.float32),
            pltpu.SemaphoreType.DMA,
        ],
    )
    def k(emb_hbm, ia_hbm, ip_hbm, in_hbm, out_hbm, ia_v, ip_v, in_v,
          a_v, p_v, n_v, out_v, sem):
        wid = lax.axis_index("s") * 2 + lax.axis_index("c")
        base = wid * TPW

        pltpu.sync_copy(ia_hbm.at[pl.ds(base, TPW)], ia_v)
        pltpu.sync_copy(ip_hbm.at[pl.ds(base, TPW)], ip_v)
        pltpu.sync_copy(in_hbm.at[pl.ds(base, TPW)], in_v)

        copies = [
            pltpu.make_async_copy(emb_hbm.at[ia_v], a_v, sem),
            pltpu.make_async_copy(emb_hbm.at[ip_v], p_v, sem),
            pltpu.make_async_copy(emb_hbm.at[in_v], n_v, sem),
        ]
        for cpy in copies:
            cpy.start()
        for cpy in copies:
            cpy.wait()

        lanes = lax.iota(jnp.int32, _L)
        perm_idx = {sh: (lanes ^ sh)[:, None] for sh in (1, 2, 4, 8)}

        def permute(v, sh):
            return lax.gather(v, perm_idx[sh], _DNUMS, (1,),
                              mode=lax.GatherScatterMode.PROMISE_IN_BOUNDS)

        G = 8

        def group(g, acc):
            t0 = g * G
            vts = []
            for j in range(G):
                t = t0 + j
                lane = None
                for c in range(D // _L):
                    sl = pl.ds(c * _L, _L)
                    a = a_v[t, sl]
                    p = p_v[t, sl]
                    n = n_v[t, sl]
                    pn = p - n
                    q = (p + n) - a - a
                    lane = pn * q if lane is None else lane + pn * q
                vts.append(lane)
            sh = 1
            while len(vts) > 1:
                nxt = []
                for i in range(0, len(vts), 2):
                    x, y = vts[i], vts[i + 1]
                    mask = (lanes & sh) == 0
                    z = jnp.where(mask, x, y)
                    u = jnp.where(mask, y, x)
                    nxt.append(z + permute(u, sh))
                vts = nxt
                sh *= 2
            r = vts[0]
            rr = r + permute(r, 8)
            return acc + jnp.maximum(rr + _MARGIN, 0.0)

        acc = lax.fori_loop(0, TPW // G, group, jnp.zeros((_L,), jnp.float32))
        acc = acc * 0.5

        for sh in (8, 4, 2, 1):
            acc = acc + permute(acc, sh)
        out_v[...] = acc
        pltpu.sync_copy(out_v, out_hbm.at[wid])

    return k


def kernel(embeddings, target, triplets):
    del target
    T = triplets.shape[0]
    B, D = embeddings.shape
    NW = 32
    TPW = T // NW
    ia = triplets[:, 0]
    ip = triplets[:, 1]
    inn = triplets[:, 2]
    partials = _triplet_kernel(T, B, D, NW, TPW)(embeddings, ia, ip, inn)
    return (jnp.sum(partials[:, 0]) / T, T)
